# baseline (speedup 1.0000x reference)
import functools

import jax
import jax.numpy as jnp
from jax import lax
from jax.experimental import pallas as pl
from jax.experimental.pallas import tpu as pltpu
from jax.experimental.pallas import tpu_sc as plsc

SEQ_LEN = 4096
N_EMBD = 2048


def _make_copy():
    try:
        info = plsc.get_sparse_core_info()
        num_cores, num_subcores = info.num_cores, info.num_subcores
    except Exception:
        num_cores, num_subcores = 2, 16
    num_workers = num_cores * num_subcores
    rows_per = SEQ_LEN // num_workers
    mesh = plsc.VectorSubcoreMesh(core_axis_name="c", subcore_axis_name="s")

    rows_per_core = SEQ_LEN // num_cores
    chunk = 16
    chunk_sc = chunk * num_subcores
    nchunks = rows_per_core // chunk_sc
    nbuf = 3

    @functools.partial(
        pl.kernel,
        mesh=mesh,
        out_type=jax.ShapeDtypeStruct((SEQ_LEN, N_EMBD), jnp.float32),
        scratch_types=(
            [pltpu.VMEM_SHARED((nbuf, chunk_sc, N_EMBD), jnp.float32)]
            + [pltpu.SemaphoreType.DMA] * (2 * nbuf)
        ),
    )
    def copy_k(emb_hbm, out_hbm, buf, *sems):
        sem_in, sem_out = sems[:nbuf], sems[nbuf:]
        s_idx = lax.axis_index("s")
        c_idx = lax.axis_index("c")
        sc_base = c_idx * rows_per_core
        tile_off = s_idx * chunk

        def cp_in(c):
            b = c % nbuf
            row = sc_base + c * chunk_sc + tile_off
            return pltpu.async_copy(
                emb_hbm.at[pl.ds(row, chunk)],
                buf.at[b, pl.ds(tile_off, chunk)], sem_in[b])

        def cp_out(c):
            b = c % nbuf
            row = sc_base + c * chunk_sc + tile_off
            return pltpu.async_copy(
                buf.at[b, pl.ds(tile_off, chunk)],
                out_hbm.at[pl.ds(row, chunk)], sem_out[b])

        ins, outs = {}, {}
        for c in range(min(nbuf, nchunks)):
            ins[c] = cp_in(c)
        for c in range(nchunks):
            ins[c].wait()
            outs[c] = cp_out(c)
            nxt = c + nbuf
            if nxt < nchunks:
                outs[c].wait()
                ins[nxt] = cp_in(nxt)
        for c in range(max(0, nchunks - nbuf), nchunks):
            outs[c].wait()

    return copy_k


_copy = _make_copy()


def kernel(x, emb_weight):
    del x
    return _copy(emb_weight)


# speedup vs baseline: 1.0250x; 1.0250x over previous
import functools

import jax
import jax.numpy as jnp
from jax import lax
from jax.experimental import pallas as pl
from jax.experimental.pallas import tpu as pltpu
from jax.experimental.pallas import tpu_sc as plsc

SEQ_LEN = 4096
N_EMBD = 2048


def _make_copy():
    try:
        info = plsc.get_sparse_core_info()
        num_cores, num_subcores = info.num_cores, info.num_subcores
    except Exception:
        num_cores, num_subcores = 2, 16
    num_workers = num_cores * num_subcores
    rows_per = SEQ_LEN // num_workers
    mesh = plsc.VectorSubcoreMesh(core_axis_name="c", subcore_axis_name="s")

    chunk = 16
    nchunks = rows_per // chunk
    nbuf = 2
    n_a = (nchunks + 1) // 2
    n_b = nchunks - n_a
    spmem_rows = chunk * num_subcores

    @functools.partial(
        pl.kernel,
        mesh=mesh,
        out_type=jax.ShapeDtypeStruct((SEQ_LEN, N_EMBD), jnp.float32),
        scratch_types=(
            [pltpu.VMEM((nbuf, chunk, N_EMBD), jnp.float32),
             pltpu.VMEM_SHARED((nbuf, spmem_rows, N_EMBD), jnp.float32)]
            + [pltpu.SemaphoreType.DMA] * (4 * nbuf)
        ),
    )
    def copy_k(emb_hbm, out_hbm, tbuf, sbuf, *sems):
        sem_ain, sem_aout = sems[:nbuf], sems[nbuf:2 * nbuf]
        sem_bin, sem_bout = sems[2 * nbuf:3 * nbuf], sems[3 * nbuf:]
        s_idx = lax.axis_index("s")
        c_idx = lax.axis_index("c")
        wid = s_idx * num_cores + c_idx
        base = wid * rows_per
        tile_off = s_idx * chunk

        def a_in(p):
            b = p % nbuf
            return pltpu.async_copy(
                emb_hbm.at[pl.ds(base + (2 * p) * chunk, chunk)],
                tbuf.at[b], sem_ain[b])

        def a_out(p):
            b = p % nbuf
            return pltpu.async_copy(
                tbuf.at[b], out_hbm.at[pl.ds(base + (2 * p) * chunk, chunk)],
                sem_aout[b])

        def b_in(p):
            b = p % nbuf
            return pltpu.async_copy(
                emb_hbm.at[pl.ds(base + (2 * p + 1) * chunk, chunk)],
                sbuf.at[b, pl.ds(tile_off, chunk)], sem_bin[b])

        def b_out(p):
            b = p % nbuf
            return pltpu.async_copy(
                sbuf.at[b, pl.ds(tile_off, chunk)],
                out_hbm.at[pl.ds(base + (2 * p + 1) * chunk, chunk)],
                sem_bout[b])

        class _Ring:
            def __init__(self, cp_in, cp_out, n):
                self.cp_in, self.cp_out, self.n = cp_in, cp_out, n
                self.ins, self.outs = {}, {}

            def prime(self):
                for p in range(min(nbuf, self.n)):
                    self.ins[p] = self.cp_in(p)

            def step(self, p):
                if p >= self.n:
                    return
                self.ins[p].wait()
                self.outs[p] = self.cp_out(p)
                nxt = p + nbuf
                if nxt < self.n:
                    self.outs[p].wait()
                    self.ins[nxt] = self.cp_in(nxt)

            def drain(self):
                for p in range(max(0, self.n - nbuf), self.n):
                    self.outs[p].wait()

        ring_a = _Ring(a_in, a_out, n_a)
        ring_b = _Ring(b_in, b_out, n_b)
        ring_a.prime()
        ring_b.prime()
        for p in range(max(n_a, n_b)):
            ring_a.step(p)
            ring_b.step(p)
        ring_a.drain()
        ring_b.drain()

    return copy_k


_copy = _make_copy()


def kernel(x, emb_weight):
    del x
    return _copy(emb_weight)


# speedup vs baseline: 1.0343x; 1.0090x over previous
import functools

import jax
import jax.numpy as jnp
from jax import lax
from jax.experimental import pallas as pl
from jax.experimental.pallas import tpu as pltpu
from jax.experimental.pallas import tpu_sc as plsc

SEQ_LEN = 4096
N_EMBD = 2048


def _make_copy():
    try:
        info = plsc.get_sparse_core_info()
        num_cores, num_subcores = info.num_cores, info.num_subcores
    except Exception:
        num_cores, num_subcores = 2, 16
    num_workers = num_cores * num_subcores
    rows_per = SEQ_LEN // num_workers
    mesh = plsc.VectorSubcoreMesh(core_axis_name="c", subcore_axis_name="s")

    chunk = 16
    nchunks = rows_per // chunk
    nbuf = 3

    @functools.partial(
        pl.kernel,
        mesh=mesh,
        out_type=jax.ShapeDtypeStruct((SEQ_LEN, N_EMBD), jnp.float32),
        scratch_types=(
            [pltpu.VMEM((nbuf, chunk, N_EMBD), jnp.float32)]
            + [pltpu.SemaphoreType.DMA] * (2 * nbuf)
        ),
    )
    def copy_k(emb_hbm, out_hbm, buf, *sems):
        sem_in, sem_out = sems[:nbuf], sems[nbuf:]
        wid = lax.axis_index("s") * num_cores + lax.axis_index("c")
        base = wid * rows_per

        def cp_in(c):
            b = c % nbuf
            return pltpu.async_copy(
                emb_hbm.at[pl.ds(base + c * chunk, chunk)], buf.at[b], sem_in[b])

        def cp_out(c):
            b = c % nbuf
            return pltpu.async_copy(
                buf.at[b], out_hbm.at[pl.ds(base + c * chunk, chunk)], sem_out[b])

        ins, outs = {}, {}
        for c in range(min(nbuf, nchunks)):
            ins[c] = cp_in(c)
        for c in range(nchunks):
            ins[c].wait()
            outs[c] = cp_out(c)
            nxt = c + nbuf
            if nxt < nchunks:
                outs[c].wait()
                ins[nxt] = cp_in(nxt)
        for c in range(max(0, nchunks - nbuf), nchunks):
            outs[c].wait()

    return copy_k


_copy = _make_copy()


def kernel(x, emb_weight):
    del x
    return _copy(emb_weight)
